# baseline (speedup 1.0000x reference)
import functools

import jax
import jax.numpy as jnp
from jax.experimental import pallas as pl
from jax.experimental.pallas import tpu as pltpu

_NT = 512


def _toeplitz(wfilt, d):
    kw = wfilt.shape[0]
    jcol = jnp.arange(d + kw - 1)[:, None]
    wcol = jnp.arange(d)[None, :]
    idx = jcol - wcol
    valid = (idx >= 0) & (idx < kw)
    idx_c = jnp.clip(idx, 0, kw - 1)
    return jnp.where(valid, wfilt[idx_c], jnp.zeros((), wfilt.dtype))


def _body(xc_ref, xp_ref, xn_ref, wk0_ref, t10_ref, t11_ref, t12_ref,
          cb_ref, e0_ref, e1_ref, emb0_ref, emb1_ref, et0_ref, et1_ref,
          gw0_ref, gw1_ref,
          zq_ref, ind_ref, inp_ref, qz_ref, *, kdim):
    nt = xc_ref.shape[0]
    dot = functools.partial(jnp.dot, preferred_element_type=jnp.float32)

    f0 = jax.nn.relu(dot(xc_ref[...], wk0_ref[...]) + cb_ref[0, 0])
    f1 = jax.nn.relu((dot(xp_ref[...], t10_ref[...])
                      + dot(xc_ref[...], t11_ref[...])
                      + dot(xn_ref[...], t12_ref[...])) + cb_ref[0, 1])
    inp_ref[:, :f0.shape[1]] = f0
    inp_ref[:, f0.shape[1]:] = f1

    iota = jax.lax.broadcasted_iota(jnp.int32, (nt, kdim), 1)

    def quantize(f, e_ref, emb_ref, et_ref, out_slice):
        dist = (jnp.sum(f * f, axis=1, keepdims=True)
                - 2.0 * dot(f, emb_ref[...])) + e_ref[...]
        mind = jnp.min(dist, axis=1, keepdims=True)
        ind = jnp.min(jnp.where(dist == mind, iota, kdim), axis=1)
        oh = (iota == ind[:, None]).astype(jnp.float32)
        q = dot(oh, et_ref[...])
        qz_ref[:, out_slice] = q
        return ind, q

    ind0, q0 = quantize(f0, e0_ref, emb0_ref, et0_ref, slice(0, f0.shape[1]))
    ind1, q1 = quantize(f1, e1_ref, emb1_ref, et1_ref,
                        slice(f0.shape[1], 2 * f0.shape[1]))
    ind_ref[:, 0:1] = ind0[:, None]
    ind_ref[:, 1:2] = ind1[:, None]

    zq0 = f0 + (q0 - f0)
    zq1 = f1 + (q1 - f1)
    g0 = jnp.sum(zq0 * gw0_ref[...], axis=1, keepdims=True) + cb_ref[0, 2]
    g1 = jnp.sum(zq0 * gw1_ref[...], axis=1, keepdims=True) + cb_ref[0, 3]
    m = jnp.maximum(g0, g1)
    a0 = jnp.exp(g0 - m)
    a1 = jnp.exp(g1 - m)
    tot = a0 + a1
    zq_ref[...] = zq0 * (a0 / tot) + zq1 * (a1 / tot)


def kernel(x, conv_w0, conv_b0, conv_w1, conv_b1, embed0, embed1,
           gate_w0, gate_b0, gate_w1, gate_b1):
    b, s, d = x.shape
    kdim = embed0.shape[1]
    n = b * s
    dp = 2 * d

    wk0 = jnp.pad(_toeplitz(conv_w0[0, 0, 0], d), ((0, 1), (0, 0)))
    t10 = jnp.pad(_toeplitz(conv_w1[0, 0, 0], d), ((0, 1), (0, 0)))
    t11 = jnp.pad(_toeplitz(conv_w1[0, 0, 1], d), ((0, 1), (0, 0)))
    t12 = jnp.pad(_toeplitz(conv_w1[0, 0, 2], d), ((0, 1), (0, 0)))
    pad_lo = (d - 1) // 2
    pad_hi = (d - 1) - pad_lo + 1
    xpad = jnp.pad(x, ((0, 0), (0, 0), (pad_lo, pad_hi)))
    xprev = jnp.pad(xpad, ((0, 0), (1, 0), (0, 0)))[:, :-1]
    xnext = jnp.pad(xpad, ((0, 0), (0, 1), (0, 0)))[:, 1:]
    xc = xpad.reshape(n, dp)
    xp = xprev.reshape(n, dp)
    xn = xnext.reshape(n, dp)

    e0 = jnp.sum(embed0 * embed0, axis=0, keepdims=True)
    e1 = jnp.sum(embed1 * embed1, axis=0, keepdims=True)
    cb = jnp.stack([conv_b0[0], conv_b1[0], gate_b0[0], gate_b1[0]])[None, :]

    grid = (n // _NT,)
    row = lambda i: (i, 0)
    whole = lambda i: (0, 0)
    out_shape = (
        jax.ShapeDtypeStruct((n, d), jnp.float32),
        jax.ShapeDtypeStruct((n, 2), jnp.int32),
        jax.ShapeDtypeStruct((n, dp), jnp.float32),
        jax.ShapeDtypeStruct((n, dp), jnp.float32),
    )
    in_specs = [
        pl.BlockSpec((_NT, dp), row),
        pl.BlockSpec((_NT, dp), row),
        pl.BlockSpec((_NT, dp), row),
        pl.BlockSpec((dp, d), whole),
        pl.BlockSpec((dp, d), whole),
        pl.BlockSpec((dp, d), whole),
        pl.BlockSpec((dp, d), whole),
        pl.BlockSpec((1, 4), whole),
        pl.BlockSpec((1, kdim), whole),
        pl.BlockSpec((1, kdim), whole),
        pl.BlockSpec((d, kdim), whole),
        pl.BlockSpec((d, kdim), whole),
        pl.BlockSpec((kdim, d), whole),
        pl.BlockSpec((kdim, d), whole),
        pl.BlockSpec((1, d), whole),
        pl.BlockSpec((1, d), whole),
    ]
    out_specs = (
        pl.BlockSpec((_NT, d), row),
        pl.BlockSpec((_NT, 2), row),
        pl.BlockSpec((_NT, dp), row),
        pl.BlockSpec((_NT, dp), row),
    )
    z_q, argmin, inputs, quantizes = pl.pallas_call(
        functools.partial(_body, kdim=kdim),
        grid=grid,
        in_specs=in_specs,
        out_specs=out_specs,
        out_shape=out_shape,
        compiler_params=pltpu.CompilerParams(
            dimension_semantics=("parallel",)),
    )(xc, xp, xn, wk0, t10, t11, t12, cb, e0, e1,
      embed0, embed1, embed0.T, embed1.T,
      gate_w0[None, :], gate_w1[None, :])
    return (z_q, argmin, inputs, quantizes)


# speedup vs baseline: 30.0389x; 30.0389x over previous
import functools

import jax
import jax.numpy as jnp
from jax.experimental import pallas as pl
from jax.experimental.pallas import tpu as pltpu

_NT = 512


def _toeplitz2d(wfilt, d):
    v_ext = jnp.concatenate([wfilt, jnp.zeros((d + 1,), wfilt.dtype)])
    b = jnp.broadcast_to(v_ext, (d, 2 * d + 1)).reshape(-1)[: 2 * d * d]
    return b.reshape(d, 2 * d).T


def _body(xc_ref, xp_ref, xn_ref, wk0_ref, t10_ref, t11_ref, t12_ref,
          cb_ref, e0_ref, e1_ref, emb0_ref, emb1_ref, et0_ref, et1_ref,
          gw0_ref, gw1_ref,
          zq_ref, ind_ref, inp_ref, qz_ref, *, kdim):
    nt = xc_ref.shape[0]
    dot = functools.partial(jnp.dot, preferred_element_type=jnp.float32)

    f0 = jax.nn.relu(dot(xc_ref[...], wk0_ref[...]) + cb_ref[0, 0])
    f1 = jax.nn.relu((dot(xp_ref[...], t10_ref[...])
                      + dot(xc_ref[...], t11_ref[...])
                      + dot(xn_ref[...], t12_ref[...])) + cb_ref[0, 1])
    inp_ref[:, :f0.shape[1]] = f0
    inp_ref[:, f0.shape[1]:] = f1

    iota = jax.lax.broadcasted_iota(jnp.int32, (nt, kdim), 1)

    def quantize(f, e_ref, emb_ref, et_ref, out_slice):
        dist = (jnp.sum(f * f, axis=1, keepdims=True)
                - 2.0 * dot(f, emb_ref[...])) + e_ref[...]
        mind = jnp.min(dist, axis=1, keepdims=True)
        ind = jnp.min(jnp.where(dist == mind, iota, kdim), axis=1)
        oh = (iota == ind[:, None]).astype(jnp.float32)
        q = dot(oh, et_ref[...])
        qz_ref[:, out_slice] = q
        return ind, q

    ind0, q0 = quantize(f0, e0_ref, emb0_ref, et0_ref, slice(0, f0.shape[1]))
    ind1, q1 = quantize(f1, e1_ref, emb1_ref, et1_ref,
                        slice(f0.shape[1], 2 * f0.shape[1]))
    ind_ref[:, 0:1] = ind0[:, None]
    ind_ref[:, 1:2] = ind1[:, None]

    zq0 = f0 + (q0 - f0)
    zq1 = f1 + (q1 - f1)
    g0 = jnp.sum(zq0 * gw0_ref[...], axis=1, keepdims=True) + cb_ref[0, 2]
    g1 = jnp.sum(zq0 * gw1_ref[...], axis=1, keepdims=True) + cb_ref[0, 3]
    m = jnp.maximum(g0, g1)
    a0 = jnp.exp(g0 - m)
    a1 = jnp.exp(g1 - m)
    tot = a0 + a1
    zq_ref[...] = zq0 * (a0 / tot) + zq1 * (a1 / tot)


def kernel(x, conv_w0, conv_b0, conv_w1, conv_b1, embed0, embed1,
           gate_w0, gate_b0, gate_w1, gate_b1):
    b, s, d = x.shape
    kdim = embed0.shape[1]
    n = b * s
    dp = 2 * d

    wk0 = _toeplitz2d(conv_w0[0, 0, 0], d)
    t10 = _toeplitz2d(conv_w1[0, 0, 0], d)
    t11 = _toeplitz2d(conv_w1[0, 0, 1], d)
    t12 = _toeplitz2d(conv_w1[0, 0, 2], d)
    pad_lo = (d - 1) // 2
    pad_hi = (d - 1) - pad_lo + 1
    xpad = jnp.pad(x, ((0, 0), (0, 0), (pad_lo, pad_hi)))
    xprev = jnp.pad(xpad, ((0, 0), (1, 0), (0, 0)))[:, :-1]
    xnext = jnp.pad(xpad, ((0, 0), (0, 1), (0, 0)))[:, 1:]
    xc = xpad.reshape(n, dp)
    xp = xprev.reshape(n, dp)
    xn = xnext.reshape(n, dp)

    e0 = jnp.sum(embed0 * embed0, axis=0, keepdims=True)
    e1 = jnp.sum(embed1 * embed1, axis=0, keepdims=True)
    cb = jnp.stack([conv_b0[0], conv_b1[0], gate_b0[0], gate_b1[0]])[None, :]

    grid = (n // _NT,)
    row = lambda i: (i, 0)
    whole = lambda i: (0, 0)
    out_shape = (
        jax.ShapeDtypeStruct((n, d), jnp.float32),
        jax.ShapeDtypeStruct((n, 2), jnp.int32),
        jax.ShapeDtypeStruct((n, dp), jnp.float32),
        jax.ShapeDtypeStruct((n, dp), jnp.float32),
    )
    in_specs = [
        pl.BlockSpec((_NT, dp), row),
        pl.BlockSpec((_NT, dp), row),
        pl.BlockSpec((_NT, dp), row),
        pl.BlockSpec((dp, d), whole),
        pl.BlockSpec((dp, d), whole),
        pl.BlockSpec((dp, d), whole),
        pl.BlockSpec((dp, d), whole),
        pl.BlockSpec((1, 4), whole),
        pl.BlockSpec((1, kdim), whole),
        pl.BlockSpec((1, kdim), whole),
        pl.BlockSpec((d, kdim), whole),
        pl.BlockSpec((d, kdim), whole),
        pl.BlockSpec((kdim, d), whole),
        pl.BlockSpec((kdim, d), whole),
        pl.BlockSpec((1, d), whole),
        pl.BlockSpec((1, d), whole),
    ]
    out_specs = (
        pl.BlockSpec((_NT, d), row),
        pl.BlockSpec((_NT, 2), row),
        pl.BlockSpec((_NT, dp), row),
        pl.BlockSpec((_NT, dp), row),
    )
    z_q, argmin, inputs, quantizes = pl.pallas_call(
        functools.partial(_body, kdim=kdim),
        grid=grid,
        in_specs=in_specs,
        out_specs=out_specs,
        out_shape=out_shape,
        compiler_params=pltpu.CompilerParams(
            dimension_semantics=("parallel",)),
    )(xc, xp, xn, wk0, t10, t11, t12, cb, e0, e1,
      embed0, embed1, embed0.T, embed1.T,
      gate_w0[None, :], gate_w1[None, :])
    return (z_q, argmin, inputs, quantizes)


# speedup vs baseline: 38.0515x; 1.2667x over previous
import functools

import jax
import jax.numpy as jnp
from jax.experimental import pallas as pl
from jax.experimental.pallas import tpu as pltpu

_NT = 512


def _toeplitz2d(wfilt, d):
    v_ext = jnp.concatenate([wfilt, jnp.zeros((d + 1,), wfilt.dtype)])
    b = jnp.broadcast_to(v_ext, (d, 2 * d + 1)).reshape(-1)[: 2 * d * d]
    t = b.reshape(d, 2 * d).T
    return jnp.concatenate([jnp.zeros((1, d), wfilt.dtype), t[:-1]])


def _body(xc_ref, pe_ref, ne_ref, wk0_ref, t10_ref, t11_ref, t12_ref,
          cb_ref, e0_ref, e1_ref, emb0_ref, emb1_ref, et0_ref, et1_ref,
          gw0_ref, gw1_ref,
          zq_ref, ind_ref, inp_ref, qz_ref, *, kdim):
    nt = xc_ref.shape[0]
    d = xc_ref.shape[1]
    hp = d // 2
    dot = functools.partial(jnp.dot, preferred_element_type=jnp.float32)

    xpq = jnp.pad(xc_ref[...], ((0, 0), (hp, hp)))
    f0 = jax.nn.relu(dot(xpq, wk0_ref[...]) + cb_ref[0, 0])
    u0 = dot(xpq, t10_ref[...])
    u1 = dot(xpq, t11_ref[...])
    u2 = dot(xpq, t12_ref[...])
    u0s = jnp.concatenate([dot(pe_ref[0], t10_ref[...]), u0[:-1]], axis=0)
    u2s = jnp.concatenate([u2[1:], dot(ne_ref[0], t12_ref[...])], axis=0)
    f1 = jax.nn.relu(((u0s + u1) + u2s) + cb_ref[0, 1])
    inp_ref[:, :d] = f0
    inp_ref[:, d:] = f1

    iota = jax.lax.broadcasted_iota(jnp.int32, (nt, kdim), 1)

    def quantize(f, e_ref, emb_ref, et_ref, out_slice):
        dist = (jnp.sum(f * f, axis=1, keepdims=True)
                - 2.0 * dot(f, emb_ref[...])) + e_ref[...]
        mind = jnp.min(dist, axis=1, keepdims=True)
        ind = jnp.min(jnp.where(dist == mind, iota, kdim), axis=1)
        oh = (iota == ind[:, None]).astype(jnp.float32)
        q = dot(oh, et_ref[...])
        qz_ref[:, out_slice] = q
        return ind, q

    ind0, q0 = quantize(f0, e0_ref, emb0_ref, et0_ref, slice(0, d))
    ind1, q1 = quantize(f1, e1_ref, emb1_ref, et1_ref, slice(d, 2 * d))
    ind_ref[:, 0:1] = ind0[:, None]
    ind_ref[:, 1:2] = ind1[:, None]

    zq0 = f0 + (q0 - f0)
    zq1 = f1 + (q1 - f1)
    g0 = jnp.sum(zq0 * gw0_ref[...], axis=1, keepdims=True) + cb_ref[0, 2]
    g1 = jnp.sum(zq0 * gw1_ref[...], axis=1, keepdims=True) + cb_ref[0, 3]
    m = jnp.maximum(g0, g1)
    a0 = jnp.exp(g0 - m)
    a1 = jnp.exp(g1 - m)
    tot = a0 + a1
    zq_ref[...] = zq0 * (a0 / tot) + zq1 * (a1 / tot)


def kernel(x, conv_w0, conv_b0, conv_w1, conv_b1, embed0, embed1,
           gate_w0, gate_b0, gate_w1, gate_b1):
    b, s, d = x.shape
    kdim = embed0.shape[1]
    n = b * s
    dp = 2 * d
    ntiles = n // _NT

    wk0 = _toeplitz2d(conv_w0[0, 0, 0], d)
    t10 = _toeplitz2d(conv_w1[0, 0, 0], d)
    t11 = _toeplitz2d(conv_w1[0, 0, 1], d)
    t12 = _toeplitz2d(conv_w1[0, 0, 2], d)

    xc = x.reshape(n, d)
    hp = d // 2
    tile_start = jnp.arange(ntiles) * _NT
    last_rows = jnp.pad(xc[_NT - 1::_NT], ((0, 0), (hp, hp)))
    first_rows = jnp.pad(xc[::_NT], ((0, 0), (hp, hp)))
    pe = jnp.concatenate([jnp.zeros((1, dp)), last_rows[:-1]])
    pe = jnp.where((tile_start % s == 0)[:, None], 0.0, pe)[:, None, :]
    ne = jnp.concatenate([first_rows[1:], jnp.zeros((1, dp))])
    ne = jnp.where(((tile_start + _NT) % s == 0)[:, None], 0.0, ne)[:, None, :]

    e0 = jnp.sum(embed0 * embed0, axis=0, keepdims=True)
    e1 = jnp.sum(embed1 * embed1, axis=0, keepdims=True)
    cb = jnp.stack([conv_b0[0], conv_b1[0], gate_b0[0], gate_b1[0]])[None, :]

    grid = (ntiles,)
    row = lambda i: (i, 0)
    whole = lambda i: (0, 0)
    edge = lambda i: (i, 0, 0)
    out_shape = (
        jax.ShapeDtypeStruct((n, d), jnp.float32),
        jax.ShapeDtypeStruct((n, 2), jnp.int32),
        jax.ShapeDtypeStruct((n, dp), jnp.float32),
        jax.ShapeDtypeStruct((n, dp), jnp.float32),
    )
    in_specs = [
        pl.BlockSpec((_NT, d), row),
        pl.BlockSpec((1, 1, dp), edge),
        pl.BlockSpec((1, 1, dp), edge),
        pl.BlockSpec((dp, d), whole),
        pl.BlockSpec((dp, d), whole),
        pl.BlockSpec((dp, d), whole),
        pl.BlockSpec((dp, d), whole),
        pl.BlockSpec((1, 4), whole),
        pl.BlockSpec((1, kdim), whole),
        pl.BlockSpec((1, kdim), whole),
        pl.BlockSpec((d, kdim), whole),
        pl.BlockSpec((d, kdim), whole),
        pl.BlockSpec((kdim, d), whole),
        pl.BlockSpec((kdim, d), whole),
        pl.BlockSpec((1, d), whole),
        pl.BlockSpec((1, d), whole),
    ]
    out_specs = (
        pl.BlockSpec((_NT, d), row),
        pl.BlockSpec((_NT, 2), row),
        pl.BlockSpec((_NT, dp), row),
        pl.BlockSpec((_NT, dp), row),
    )
    z_q, argmin, inputs, quantizes = pl.pallas_call(
        functools.partial(_body, kdim=kdim),
        grid=grid,
        in_specs=in_specs,
        out_specs=out_specs,
        out_shape=out_shape,
        compiler_params=pltpu.CompilerParams(
            dimension_semantics=("parallel",)),
    )(xc, pe, ne, wk0, t10, t11, t12, cb, e0, e1,
      embed0, embed1, embed0.T, embed1.T,
      gate_w0[None, :], gate_w1[None, :])
    return (z_q, argmin, inputs, quantizes)


# speedup vs baseline: 41.0344x; 1.0784x over previous
import functools

import jax
import jax.numpy as jnp
from jax.experimental import pallas as pl
from jax.experimental.pallas import tpu as pltpu

_NT = 1024


def _toeplitz2d(wfilt, d):
    v_ext = jnp.concatenate([wfilt, jnp.zeros((d + 1,), wfilt.dtype)])
    b = jnp.broadcast_to(v_ext, (d, 2 * d + 1)).reshape(-1)[: 2 * d * d]
    t = b.reshape(d, 2 * d).T
    return jnp.concatenate([jnp.zeros((1, d), wfilt.dtype), t[:-1]])


def _body(xc_ref, pe_ref, ne_ref, wk0_ref, t10_ref, t11_ref, t12_ref,
          cb_ref, e0_ref, e1_ref, emb0_ref, emb1_ref, et0_ref, et1_ref,
          gw0_ref, gw1_ref,
          zq_ref, ind_ref, inp_ref, qz_ref, *, kdim):
    nt = xc_ref.shape[0]
    d = xc_ref.shape[1]
    hp = d // 2
    dot = functools.partial(jnp.dot, preferred_element_type=jnp.float32)

    xpq = jnp.pad(xc_ref[...], ((0, 0), (hp, hp)))
    f0 = jax.nn.relu(dot(xpq, wk0_ref[...]) + cb_ref[0, 0])
    u0 = dot(xpq, t10_ref[...])
    u1 = dot(xpq, t11_ref[...])
    u2 = dot(xpq, t12_ref[...])
    u0s = jnp.concatenate([dot(pe_ref[0], t10_ref[...]), u0[:-1]], axis=0)
    u2s = jnp.concatenate([u2[1:], dot(ne_ref[0], t12_ref[...])], axis=0)
    f1 = jax.nn.relu(((u0s + u1) + u2s) + cb_ref[0, 1])
    inp_ref[:, :d] = f0
    inp_ref[:, d:] = f1

    iota = jax.lax.broadcasted_iota(jnp.int32, (nt, kdim), 1)

    def quantize(f, e_ref, emb_ref, et_ref, out_slice):
        dist = (jnp.sum(f * f, axis=1, keepdims=True)
                - 2.0 * dot(f, emb_ref[...])) + e_ref[...]
        ind = jnp.argmin(dist, axis=1).astype(jnp.int32)
        oh = (iota == ind[:, None]).astype(jnp.bfloat16)
        q = dot(oh, et_ref[...])
        qz_ref[:, out_slice] = q
        return ind, q

    ind0, q0 = quantize(f0, e0_ref, emb0_ref, et0_ref, slice(0, d))
    ind1, q1 = quantize(f1, e1_ref, emb1_ref, et1_ref, slice(d, 2 * d))
    ind_ref[:, 0:1] = ind0[:, None]
    ind_ref[:, 1:2] = ind1[:, None]

    zq0 = f0 + (q0 - f0)
    zq1 = f1 + (q1 - f1)
    g0 = jnp.sum(zq0 * gw0_ref[...], axis=1, keepdims=True) + cb_ref[0, 2]
    g1 = jnp.sum(zq0 * gw1_ref[...], axis=1, keepdims=True) + cb_ref[0, 3]
    m = jnp.maximum(g0, g1)
    a0 = jnp.exp(g0 - m)
    a1 = jnp.exp(g1 - m)
    tot = a0 + a1
    zq_ref[...] = zq0 * (a0 / tot) + zq1 * (a1 / tot)


def kernel(x, conv_w0, conv_b0, conv_w1, conv_b1, embed0, embed1,
           gate_w0, gate_b0, gate_w1, gate_b1):
    b, s, d = x.shape
    kdim = embed0.shape[1]
    n = b * s
    dp = 2 * d
    ntiles = n // _NT

    wk0 = _toeplitz2d(conv_w0[0, 0, 0], d)
    t10 = _toeplitz2d(conv_w1[0, 0, 0], d)
    t11 = _toeplitz2d(conv_w1[0, 0, 1], d)
    t12 = _toeplitz2d(conv_w1[0, 0, 2], d)

    xc = x.reshape(n, d)
    hp = d // 2
    tile_start = jnp.arange(ntiles) * _NT
    last_rows = jnp.pad(xc[_NT - 1::_NT], ((0, 0), (hp, hp)))
    first_rows = jnp.pad(xc[::_NT], ((0, 0), (hp, hp)))
    pe = jnp.concatenate([jnp.zeros((1, dp)), last_rows[:-1]])
    pe = jnp.where((tile_start % s == 0)[:, None], 0.0, pe)[:, None, :]
    ne = jnp.concatenate([first_rows[1:], jnp.zeros((1, dp))])
    ne = jnp.where(((tile_start + _NT) % s == 0)[:, None], 0.0, ne)[:, None, :]

    e0 = jnp.sum(embed0 * embed0, axis=0, keepdims=True)
    e1 = jnp.sum(embed1 * embed1, axis=0, keepdims=True)
    cb = jnp.stack([conv_b0[0], conv_b1[0], gate_b0[0], gate_b1[0]])[None, :]

    grid = (ntiles,)
    row = lambda i: (i, 0)
    whole = lambda i: (0, 0)
    edge = lambda i: (i, 0, 0)
    out_shape = (
        jax.ShapeDtypeStruct((n, d), jnp.float32),
        jax.ShapeDtypeStruct((n, 2), jnp.int32),
        jax.ShapeDtypeStruct((n, dp), jnp.float32),
        jax.ShapeDtypeStruct((n, dp), jnp.float32),
    )
    in_specs = [
        pl.BlockSpec((_NT, d), row),
        pl.BlockSpec((1, 1, dp), edge),
        pl.BlockSpec((1, 1, dp), edge),
        pl.BlockSpec((dp, d), whole),
        pl.BlockSpec((dp, d), whole),
        pl.BlockSpec((dp, d), whole),
        pl.BlockSpec((dp, d), whole),
        pl.BlockSpec((1, 4), whole),
        pl.BlockSpec((1, kdim), whole),
        pl.BlockSpec((1, kdim), whole),
        pl.BlockSpec((d, kdim), whole),
        pl.BlockSpec((d, kdim), whole),
        pl.BlockSpec((kdim, d), whole),
        pl.BlockSpec((kdim, d), whole),
        pl.BlockSpec((1, d), whole),
        pl.BlockSpec((1, d), whole),
    ]
    out_specs = (
        pl.BlockSpec((_NT, d), row),
        pl.BlockSpec((_NT, 2), row),
        pl.BlockSpec((_NT, dp), row),
        pl.BlockSpec((_NT, dp), row),
    )
    z_q, argmin, inputs, quantizes = pl.pallas_call(
        functools.partial(_body, kdim=kdim),
        grid=grid,
        in_specs=in_specs,
        out_specs=out_specs,
        out_shape=out_shape,
        compiler_params=pltpu.CompilerParams(
            dimension_semantics=("parallel",)),
    )(xc, pe, ne, wk0, t10, t11, t12, cb, e0, e1,
      embed0, embed1,
      embed0.T.astype(jnp.bfloat16), embed1.T.astype(jnp.bfloat16),
      gate_w0[None, :], gate_w1[None, :])
    return (z_q, argmin, inputs, quantizes)


# speedup vs baseline: 49.0836x; 1.1962x over previous
import functools

import jax
import jax.numpy as jnp
from jax.experimental import pallas as pl
from jax.experimental.pallas import tpu as pltpu

_NT = 1024

_TDOT = (((1,), (1,)), ((), ()))


def _toeplitz_stack(filts, d):
    v_ext = jnp.concatenate([jnp.zeros((4, 1), filts.dtype), filts,
                             jnp.zeros((4, d), filts.dtype)], axis=1)
    b = jnp.broadcast_to(v_ext[:, None, :], (4, d, 2 * d + 1))
    return b.reshape(4, -1)[:, : 2 * d * d].reshape(4, d, 2 * d)


def _body(xc_ref, xm_ref, xp_ref, b4_ref, cb_ref, e0_ref, e1_ref,
          emb0_ref, emb1_ref, ebf0_ref, ebf1_ref, gw0_ref, gw1_ref,
          zq_ref, ind_ref, inp_ref, qz_ref, *, kdim, nseq):
    nt = xc_ref.shape[0]
    d = xc_ref.shape[1]
    hp = d // 2
    i = pl.program_id(0)
    tdot = functools.partial(jax.lax.dot_general, dimension_numbers=_TDOT,
                             preferred_element_type=jnp.float32)

    xpq = jnp.pad(xc_ref[...], ((0, 0), (hp, hp)))
    f0 = jax.nn.relu(tdot(xpq, b4_ref[0]) + cb_ref[0, 0])
    u0 = tdot(xpq, b4_ref[1])
    u1 = tdot(xpq, b4_ref[2])
    u2 = tdot(xpq, b4_ref[3])
    tiles_per_seq = nseq // nt
    pe_row = jnp.pad(xm_ref[nt - 1:nt, :], ((0, 0), (hp, hp)))
    pe_row = jnp.where(i % tiles_per_seq == 0, 0.0, pe_row)
    ne_row = jnp.pad(xp_ref[0:1, :], ((0, 0), (hp, hp)))
    ne_row = jnp.where((i + 1) % tiles_per_seq == 0, 0.0, ne_row)
    u0s = jnp.concatenate([tdot(pe_row, b4_ref[1]), u0[:-1]], axis=0)
    u2s = jnp.concatenate([u2[1:], tdot(ne_row, b4_ref[3])], axis=0)
    f1 = jax.nn.relu(((u0s + u1) + u2s) + cb_ref[0, 1])
    inp_ref[:, :d] = f0
    inp_ref[:, d:] = f1

    iota = jax.lax.broadcasted_iota(jnp.int32, (nt, kdim), 1)

    def quantize(f, e_ref, emb_ref, ebf_ref, out_slice):
        dist = (jnp.sum(f * f, axis=1, keepdims=True)
                - 2.0 * jnp.dot(f, emb_ref[...],
                                preferred_element_type=jnp.float32)) + e_ref[...]
        ind = jnp.argmin(dist, axis=1).astype(jnp.int32)
        oh = (iota == ind[:, None]).astype(jnp.bfloat16)
        q = tdot(oh, ebf_ref[...])
        qz_ref[:, out_slice] = q
        return ind, q

    ind0, q0 = quantize(f0, e0_ref, emb0_ref, ebf0_ref, slice(0, d))
    ind1, q1 = quantize(f1, e1_ref, emb1_ref, ebf1_ref, slice(d, 2 * d))
    ind_ref[:, 0:1] = ind0[:, None]
    ind_ref[:, 1:2] = ind1[:, None]

    zq0 = f0 + (q0 - f0)
    zq1 = f1 + (q1 - f1)
    g0 = jnp.sum(zq0 * gw0_ref[...], axis=1, keepdims=True) + cb_ref[0, 2]
    g1 = jnp.sum(zq0 * gw1_ref[...], axis=1, keepdims=True) + cb_ref[0, 3]
    m = jnp.maximum(g0, g1)
    a0 = jnp.exp(g0 - m)
    a1 = jnp.exp(g1 - m)
    tot = a0 + a1
    zq_ref[...] = zq0 * (a0 / tot) + zq1 * (a1 / tot)


def kernel(x, conv_w0, conv_b0, conv_w1, conv_b1, embed0, embed1,
           gate_w0, gate_b0, gate_w1, gate_b1):
    b, s, d = x.shape
    kdim = embed0.shape[1]
    n = b * s
    dp = 2 * d
    ntiles = n // _NT

    filts = jnp.concatenate([conv_w0[0, 0], conv_w1[0, 0]], axis=0)
    b4 = _toeplitz_stack(filts, d)

    xc = x.reshape(n, d)
    e0 = jnp.sum(embed0 * embed0, axis=0, keepdims=True)
    e1 = jnp.sum(embed1 * embed1, axis=0, keepdims=True)
    cb = jnp.stack([conv_b0[0], conv_b1[0], gate_b0[0], gate_b1[0]])[None, :]

    grid = (ntiles,)
    row = lambda i: (i, 0)
    prow = lambda i: (jnp.maximum(i - 1, 0), 0)
    nrow = lambda i: (jnp.minimum(i + 1, ntiles - 1), 0)
    whole = lambda i: (0, 0)
    whole3 = lambda i: (0, 0, 0)
    out_shape = (
        jax.ShapeDtypeStruct((n, d), jnp.float32),
        jax.ShapeDtypeStruct((n, 2), jnp.int32),
        jax.ShapeDtypeStruct((n, dp), jnp.float32),
        jax.ShapeDtypeStruct((n, dp), jnp.float32),
    )
    in_specs = [
        pl.BlockSpec((_NT, d), row),
        pl.BlockSpec((_NT, d), prow),
        pl.BlockSpec((_NT, d), nrow),
        pl.BlockSpec((4, d, dp), whole3),
        pl.BlockSpec((1, 4), whole),
        pl.BlockSpec((1, kdim), whole),
        pl.BlockSpec((1, kdim), whole),
        pl.BlockSpec((d, kdim), whole),
        pl.BlockSpec((d, kdim), whole),
        pl.BlockSpec((d, kdim), whole),
        pl.BlockSpec((d, kdim), whole),
        pl.BlockSpec((1, d), whole),
        pl.BlockSpec((1, d), whole),
    ]
    out_specs = (
        pl.BlockSpec((_NT, d), row),
        pl.BlockSpec((_NT, 2), row),
        pl.BlockSpec((_NT, dp), row),
        pl.BlockSpec((_NT, dp), row),
    )
    z_q, argmin, inputs, quantizes = pl.pallas_call(
        functools.partial(_body, kdim=kdim, nseq=s),
        grid=grid,
        in_specs=in_specs,
        out_specs=out_specs,
        out_shape=out_shape,
        compiler_params=pltpu.CompilerParams(
            dimension_semantics=("arbitrary",)),
    )(xc, xc, xc, b4, cb, e0, e1, embed0, embed1,
      embed0.astype(jnp.bfloat16), embed1.astype(jnp.bfloat16),
      gate_w0[None, :], gate_w1[None, :])
    return (z_q, argmin, inputs, quantizes)
